# baseline (speedup 1.0000x reference)
import jax
import jax.numpy as jnp
from jax import lax
from jax.experimental import pallas as pl
from jax.experimental.pallas import tpu as pltpu
from jax.experimental.pallas import tpu_sc as plsc

B = 16384
N_SPARSE = 26
N_DENSE = 13
VOCAB = 100000
EDIM = 16
HID1 = 256
HID2 = 128
N_FEAT = N_SPARSE + N_DENSE
SPARSE_DIM = N_SPARSE * EDIM

NC = 2
NSUB = 16
NW = NC * NSUB
ROWS_W = B // NW
CH = 128
CPF = ROWS_W // CH
NCH = N_SPARSE * CPF

TB = 512


def _sc_gather_body(idx_hbm, tab_hbm, out_hbm, idx_v, rows_v, sem0, sem1):
    wid = lax.axis_index("s") * NC + lax.axis_index("c")
    row0 = wid * ROWS_W
    sems = (sem0, sem1)

    pltpu.sync_copy(idx_hbm.at[:, pl.ds(row0, ROWS_W)], idx_v)

    def start(c, slot):
        f = c // CPF
        j = c % CPF
        pltpu.async_copy(tab_hbm.at[f].at[idx_v.at[f, pl.ds(j * CH, CH)]],
                         rows_v.at[slot], sems[slot])

    def wait(c, slot):
        f = c // CPF
        j = c % CPF
        pltpu.make_async_copy(tab_hbm.at[f].at[idx_v.at[f, pl.ds(j * CH, CH)]],
                              rows_v.at[slot], sems[slot]).wait()

    def store(c, slot):
        f = c // CPF
        j = c % CPF
        pltpu.sync_copy(rows_v.at[slot],
                        out_hbm.at[pl.ds(row0 + j * CH, CH),
                                   pl.ds(f * EDIM, EDIM)])

    start(0, 0)
    start(1, 1)

    def pair(i, carry):
        c0 = i * 2
        for b in range(2):
            c = c0 + b
            wait(c, b)
            store(c, b)

            @pl.when(c + 2 < NCH)
            def _():
                start(c + 2, b)
        return carry
    lax.fori_loop(0, NCH // 2, pair, 0)


def _sc_gather(idx_t, tables):
    mesh = plsc.VectorSubcoreMesh(core_axis_name="c", subcore_axis_name="s",
                                  num_cores=NC, num_subcores=NSUB)
    f = pl.kernel(
        _sc_gather_body,
        out_type=jax.ShapeDtypeStruct((B, SPARSE_DIM), jnp.float32),
        mesh=mesh,
        compiler_params=pltpu.CompilerParams(use_tc_tiling_on_sc=False),
        scratch_types=[
            pltpu.VMEM((N_SPARSE, ROWS_W), jnp.int32),
            pltpu.VMEM((2, CH, EDIM), jnp.float32),
            pltpu.SemaphoreType.DMA,
            pltpu.SemaphoreType.DMA,
        ],
    )
    return f(idx_t, tables)


def _mlp_body(emb_ref, x_ref, wlin_ref, blin_ref, w1e_ref, w1d_ref, b1_ref,
              w2_ref, b2_ref, wout_ref, o_ref):
    hi = jax.lax.Precision.HIGHEST
    x = x_ref[...]
    emb = emb_ref[...]
    xd = x[:, N_SPARSE:]

    wide = jnp.maximum(
        jnp.dot(x, wlin_ref[...], precision=hi,
                preferred_element_type=jnp.float32) + blin_ref[...], 0.0)

    h1 = jnp.dot(emb, w1e_ref[...], precision=hi,
                 preferred_element_type=jnp.float32)
    h1 = h1 + jnp.dot(xd, w1d_ref[...], precision=hi,
                      preferred_element_type=jnp.float32)
    h1 = jnp.maximum(h1 + b1_ref[...], 0.0)

    h2 = jnp.maximum(
        jnp.dot(h1, w2_ref[...], precision=hi,
                preferred_element_type=jnp.float32) + b2_ref[...], 0.0)

    z = wide + jnp.dot(h2, wout_ref[...], precision=hi,
                       preferred_element_type=jnp.float32)
    o_ref[...] = 1.0 / (1.0 + jnp.exp(-z))


def _mlp(emb, X, wlinT, blin, w1eT, w1dT, b1, w2T, b2, woutT):
    full = lambda shape: pl.BlockSpec(shape, lambda i: (0, 0))
    return pl.pallas_call(
        _mlp_body,
        grid=(B // TB,),
        in_specs=[
            pl.BlockSpec((TB, SPARSE_DIM), lambda i: (i, 0)),
            pl.BlockSpec((TB, N_FEAT), lambda i: (i, 0)),
            full((N_FEAT, 1)),
            full((1, 1)),
            full((SPARSE_DIM, HID1)),
            full((N_DENSE, HID1)),
            full((1, HID1)),
            full((HID1, HID2)),
            full((1, HID2)),
            full((HID2, 1)),
        ],
        out_specs=pl.BlockSpec((TB, 1), lambda i: (i, 0)),
        out_shape=jax.ShapeDtypeStruct((B, 1), jnp.float32),
    )(emb, X, wlinT, blin, w1eT, w1dT, b1, w2T, b2, woutT)


def kernel(X, tables, W_lin, b_lin, W1, b1, W2, b2, W_out):
    idx_t = X[:, :N_SPARSE].astype(jnp.int32).T

    emb = _sc_gather(idx_t, tables)

    y = _mlp(emb, X,
             W_lin.T, b_lin.reshape(1, 1),
             W1[:, :SPARSE_DIM].T, W1[:, SPARSE_DIM:].T, b1.reshape(1, HID1),
             W2.T, b2.reshape(1, HID2),
             W_out.T)
    return y


# speedup vs baseline: 4.1282x; 4.1282x over previous
import jax
import jax.numpy as jnp
from jax import lax
from jax.experimental import pallas as pl
from jax.experimental.pallas import tpu as pltpu
from jax.experimental.pallas import tpu_sc as plsc

B = 16384
N_SPARSE = 26
N_DENSE = 13
VOCAB = 100000
EDIM = 16
HID1 = 256
HID2 = 128
N_FEAT = N_SPARSE + N_DENSE
SPARSE_DIM = N_SPARSE * EDIM

NC = 2
NSUB = 16
NW = NC * NSUB
PPW = SPARSE_DIM // NW
ICH = 2048
NICH = B // ICH

TB = 512


def _sc_gather_body(idx_hbm, tab_hbm, out_hbm, vec_v, idx_v, row_v,
                    ssem, osem0, osem1):
    wid = lax.axis_index("s") * NC + lax.axis_index("c")
    osems = (osem0, osem1)

    def per_pair(k, carry):
        p = wid * PPW + k
        f = p // EDIM
        pltpu.async_copy(tab_hbm.at[p], vec_v, ssem).wait()

        def per_block_pair(jj, carry2):
            for s in range(2):
                j = jj * 2 + s
                b0 = j * ICH
                pltpu.sync_copy(idx_hbm.at[f, pl.ds(b0, ICH)], idx_v.at[s])

                def gather16(i, carry3, s=s):
                    sl = pl.ds(i * 16, 16)
                    iv = idx_v[s, sl]
                    row_v[s, sl] = plsc.load_gather(vec_v, [iv])
                    return carry3
                lax.fori_loop(0, ICH // 16, gather16, 0)

                @pl.when(j >= 2)
                def _(s=s, b0=b0):
                    pltpu.make_async_copy(
                        row_v.at[s], out_hbm.at[p, pl.ds(b0 - 2 * ICH, ICH)],
                        osems[s]).wait()
                pltpu.async_copy(row_v.at[s], out_hbm.at[p, pl.ds(b0, ICH)],
                                 osems[s])
            return carry2
        lax.fori_loop(0, NICH // 2, per_block_pair, 0)

        for jb in range(2):
            b_last = (NICH - 2 + jb) * ICH
            pltpu.make_async_copy(
                row_v.at[jb], out_hbm.at[p, pl.ds(b_last, ICH)],
                osems[jb]).wait()
        return carry
    lax.fori_loop(0, PPW, per_pair, 0)


def _sc_gather(idx_t, tab2):
    mesh = plsc.VectorSubcoreMesh(core_axis_name="c", subcore_axis_name="s",
                                  num_cores=NC, num_subcores=NSUB)
    f = pl.kernel(
        _sc_gather_body,
        out_type=jax.ShapeDtypeStruct((SPARSE_DIM, B), jnp.float32),
        mesh=mesh,
        compiler_params=pltpu.CompilerParams(use_tc_tiling_on_sc=True,
                                             needs_layout_passes=False),
        scratch_types=[
            pltpu.VMEM((VOCAB,), jnp.float32),
            pltpu.VMEM((2, ICH), jnp.int32),
            pltpu.VMEM((2, ICH), jnp.float32),
            pltpu.SemaphoreType.DMA,
            pltpu.SemaphoreType.DMA,
            pltpu.SemaphoreType.DMA,
        ],
    )
    return f(idx_t, tab2)


def _mlp_body(embT_ref, xT_ref, wlin_ref, blin_ref, w1e_ref, w1d_ref, b1_ref,
              w2_ref, b2_ref, wout_ref, o_ref):
    hi = jax.lax.Precision.HIGHEST
    xT = xT_ref[...]
    embT = embT_ref[...]
    xdT = xT[N_SPARSE:, :]

    wide = jnp.maximum(
        jnp.dot(wlin_ref[...], xT, precision=hi,
                preferred_element_type=jnp.float32) + blin_ref[...], 0.0)

    h1 = jnp.dot(w1e_ref[...], embT, precision=hi,
                 preferred_element_type=jnp.float32)
    h1 = h1 + jnp.dot(w1d_ref[...], xdT, precision=hi,
                      preferred_element_type=jnp.float32)
    h1 = jnp.maximum(h1 + b1_ref[...], 0.0)

    h2 = jnp.maximum(
        jnp.dot(w2_ref[...], h1, precision=hi,
                preferred_element_type=jnp.float32) + b2_ref[...], 0.0)

    z = wide + jnp.dot(wout_ref[...], h2, precision=hi,
                       preferred_element_type=jnp.float32)
    o_ref[...] = 1.0 / (1.0 + jnp.exp(-z))


def _mlp(embT, XT, wlin, blin, w1e, w1d, b1, w2, b2, wout):
    full = lambda shape: pl.BlockSpec(shape, lambda i: (0, 0))
    return pl.pallas_call(
        _mlp_body,
        grid=(B // TB,),
        in_specs=[
            pl.BlockSpec((SPARSE_DIM, TB), lambda i: (0, i)),
            pl.BlockSpec((N_FEAT, TB), lambda i: (0, i)),
            full((1, N_FEAT)),
            full((1, 1)),
            full((HID1, SPARSE_DIM)),
            full((HID1, N_DENSE)),
            full((HID1, 1)),
            full((HID2, HID1)),
            full((HID2, 1)),
            full((1, HID2)),
        ],
        out_specs=pl.BlockSpec((1, TB), lambda i: (0, i)),
        out_shape=jax.ShapeDtypeStruct((1, B), jnp.float32),
    )(embT, XT, wlin, blin, w1e, w1d, b1, w2, b2, wout)


def kernel(X, tables, W_lin, b_lin, W1, b1, W2, b2, W_out):
    idx_t = X[:, :N_SPARSE].astype(jnp.int32).T
    tab2 = jnp.swapaxes(tables, 1, 2).reshape(SPARSE_DIM, VOCAB)

    embT = _sc_gather(idx_t, tab2)

    XT = X.T
    y = _mlp(embT, XT,
             W_lin, b_lin.reshape(1, 1),
             W1[:, :SPARSE_DIM], W1[:, SPARSE_DIM:], b1.reshape(HID1, 1),
             W2, b2.reshape(HID2, 1),
             W_out)
    return y.reshape(B, 1)


# speedup vs baseline: 4.5655x; 1.1059x over previous
import jax
import jax.numpy as jnp
from jax import lax
from jax.experimental import pallas as pl
from jax.experimental.pallas import tpu as pltpu
from jax.experimental.pallas import tpu_sc as plsc

B = 16384
N_SPARSE = 26
N_DENSE = 13
VOCAB = 100000
EDIM = 16
HID1 = 256
HID2 = 128
N_FEAT = N_SPARSE + N_DENSE
SPARSE_DIM = N_SPARSE * EDIM

NC = 2
NSUB = 16
NW = NC * NSUB
PPW = SPARSE_DIM // NW
ICH = 2048
NICH = B // ICH

TB = 1024


def _sc_gather_body(idx_hbm, tab_hbm, out_hbm, vec_v, idx_v, row_v,
                    ssem, osem0, osem1):
    wid = lax.axis_index("s") * NC + lax.axis_index("c")
    osems = (osem0, osem1)

    def per_pair(k, carry):
        p = wid * PPW + k
        f = p // EDIM
        pltpu.async_copy(tab_hbm.at[p], vec_v, ssem).wait()

        def per_block_pair(jj, carry2):
            for s in range(2):
                j = jj * 2 + s
                b0 = j * ICH
                pltpu.sync_copy(idx_hbm.at[f, pl.ds(b0, ICH)], idx_v.at[s])

                def gather128(i, carry3, s=s):
                    for u in range(8):
                        sl = pl.ds(i * 128 + u * 16, 16)
                        iv = idx_v[s, sl]
                        row_v[s, sl] = plsc.load_gather(vec_v, [iv])
                    return carry3
                lax.fori_loop(0, ICH // 128, gather128, 0)

                @pl.when(j >= 2)
                def _(s=s, b0=b0):
                    pltpu.make_async_copy(
                        row_v.at[s], out_hbm.at[p, pl.ds(b0 - 2 * ICH, ICH)],
                        osems[s]).wait()
                pltpu.async_copy(row_v.at[s], out_hbm.at[p, pl.ds(b0, ICH)],
                                 osems[s])
            return carry2
        lax.fori_loop(0, NICH // 2, per_block_pair, 0)

        for jb in range(2):
            b_last = (NICH - 2 + jb) * ICH
            pltpu.make_async_copy(
                row_v.at[jb], out_hbm.at[p, pl.ds(b_last, ICH)],
                osems[jb]).wait()
        return carry
    lax.fori_loop(0, PPW, per_pair, 0)


def _sc_gather(idx_t, tab2):
    mesh = plsc.VectorSubcoreMesh(core_axis_name="c", subcore_axis_name="s",
                                  num_cores=NC, num_subcores=NSUB)
    f = pl.kernel(
        _sc_gather_body,
        out_type=jax.ShapeDtypeStruct((SPARSE_DIM, B), jnp.float32),
        mesh=mesh,
        compiler_params=pltpu.CompilerParams(use_tc_tiling_on_sc=True,
                                             needs_layout_passes=False),
        scratch_types=[
            pltpu.VMEM((VOCAB,), jnp.float32),
            pltpu.VMEM((2, ICH), jnp.int32),
            pltpu.VMEM((2, ICH), jnp.float32),
            pltpu.SemaphoreType.DMA,
            pltpu.SemaphoreType.DMA,
            pltpu.SemaphoreType.DMA,
        ],
    )
    return f(idx_t, tab2)


def _mlp_body(embT_ref, xT_ref, wlin_ref, blin_ref, w1e_ref, w1d_ref, b1_ref,
              w2_ref, b2_ref, wout_ref, o_ref):
    hi = jax.lax.Precision.HIGHEST
    xT = xT_ref[...]
    embT = embT_ref[...]
    xdT = xT[N_SPARSE:, :]

    wide = jnp.maximum(
        jnp.dot(wlin_ref[...], xT, precision=hi,
                preferred_element_type=jnp.float32) + blin_ref[...], 0.0)

    h1 = jnp.dot(w1e_ref[...], embT, precision=hi,
                 preferred_element_type=jnp.float32)
    h1 = h1 + jnp.dot(w1d_ref[...], xdT, precision=hi,
                      preferred_element_type=jnp.float32)
    h1 = jnp.maximum(h1 + b1_ref[...], 0.0)

    h2 = jnp.maximum(
        jnp.dot(w2_ref[...], h1, precision=hi,
                preferred_element_type=jnp.float32) + b2_ref[...], 0.0)

    z = wide + jnp.dot(wout_ref[...], h2, precision=hi,
                       preferred_element_type=jnp.float32)
    o_ref[...] = 1.0 / (1.0 + jnp.exp(-z))


def _mlp(embT, XT, wlin, blin, w1e, w1d, b1, w2, b2, wout):
    full = lambda shape: pl.BlockSpec(shape, lambda i: (0, 0))
    return pl.pallas_call(
        _mlp_body,
        grid=(B // TB,),
        in_specs=[
            pl.BlockSpec((SPARSE_DIM, TB), lambda i: (0, i)),
            pl.BlockSpec((N_FEAT, TB), lambda i: (0, i)),
            full((1, N_FEAT)),
            full((1, 1)),
            full((HID1, SPARSE_DIM)),
            full((HID1, N_DENSE)),
            full((HID1, 1)),
            full((HID2, HID1)),
            full((HID2, 1)),
            full((1, HID2)),
        ],
        out_specs=pl.BlockSpec((1, TB), lambda i: (0, i)),
        out_shape=jax.ShapeDtypeStruct((1, B), jnp.float32),
    )(embT, XT, wlin, blin, w1e, w1d, b1, w2, b2, wout)


def kernel(X, tables, W_lin, b_lin, W1, b1, W2, b2, W_out):
    idx_t = X[:, :N_SPARSE].astype(jnp.int32).T
    tab2 = jnp.swapaxes(tables, 1, 2).reshape(SPARSE_DIM, VOCAB)

    embT = _sc_gather(idx_t, tab2)

    XT = X.T
    y = _mlp(embT, XT,
             W_lin, b_lin.reshape(1, 1),
             W1[:, :SPARSE_DIM], W1[:, SPARSE_DIM:], b1.reshape(HID1, 1),
             W2, b2.reshape(HID2, 1),
             W_out)
    return y.reshape(B, 1)


# speedup vs baseline: 5.2449x; 1.1488x over previous
import jax
import jax.numpy as jnp
from jax import lax
from jax.experimental import pallas as pl
from jax.experimental.pallas import tpu as pltpu
from jax.experimental.pallas import tpu_sc as plsc

B = 16384
N_SPARSE = 26
N_DENSE = 13
VOCAB = 100000
EDIM = 16
HID1 = 256
HID2 = 128
N_FEAT = N_SPARSE + N_DENSE
SPARSE_DIM = N_SPARSE * EDIM

NC = 2
NSUB = 16
NW = NC * NSUB
PPW = SPARSE_DIM // NW
ICH = 2048
NICH = B // ICH

TB = 1024


def _sc_gather_body(idx_hbm, tab_hbm, out_hbm, vec_v, idx_v, row_v,
                    ssem, osem0, osem1):
    wid = lax.axis_index("s") * NC + lax.axis_index("c")
    osems = (osem0, osem1)

    def per_pair(k, carry):
        p = wid * PPW + k
        f = p // EDIM
        pltpu.async_copy(tab_hbm.at[p], vec_v, ssem).wait()

        def per_block_pair(jj, carry2):
            for s in range(2):
                j = jj * 2 + s
                b0 = j * ICH
                pltpu.sync_copy(idx_hbm.at[f, pl.ds(b0, ICH)], idx_v.at[s])

                @plsc.parallel_loop(0, ICH, step=16, unroll=8)
                def _gather(i, s=s):
                    sl = pl.ds(i, 16)
                    iv = idx_v[s, sl]
                    row_v[s, sl] = plsc.load_gather(vec_v, [iv])

                @pl.when(j >= 2)
                def _(s=s, b0=b0):
                    pltpu.make_async_copy(
                        row_v.at[s], out_hbm.at[p, pl.ds(b0 - 2 * ICH, ICH)],
                        osems[s]).wait()
                pltpu.async_copy(row_v.at[s], out_hbm.at[p, pl.ds(b0, ICH)],
                                 osems[s])
            return carry2
        lax.fori_loop(0, NICH // 2, per_block_pair, 0)

        for jb in range(2):
            b_last = (NICH - 2 + jb) * ICH
            pltpu.make_async_copy(
                row_v.at[jb], out_hbm.at[p, pl.ds(b_last, ICH)],
                osems[jb]).wait()
        return carry
    lax.fori_loop(0, PPW, per_pair, 0)


def _sc_gather(idx_t, tab2):
    mesh = plsc.VectorSubcoreMesh(core_axis_name="c", subcore_axis_name="s",
                                  num_cores=NC, num_subcores=NSUB)
    f = pl.kernel(
        _sc_gather_body,
        out_type=jax.ShapeDtypeStruct((SPARSE_DIM, B), jnp.float32),
        mesh=mesh,
        compiler_params=pltpu.CompilerParams(use_tc_tiling_on_sc=True,
                                             needs_layout_passes=False),
        scratch_types=[
            pltpu.VMEM((VOCAB,), jnp.float32),
            pltpu.VMEM((2, ICH), jnp.int32),
            pltpu.VMEM((2, ICH), jnp.float32),
            pltpu.SemaphoreType.DMA,
            pltpu.SemaphoreType.DMA,
            pltpu.SemaphoreType.DMA,
        ],
    )
    return f(idx_t, tab2)


def _mlp_body(embT_ref, xT_ref, wlin_ref, blin_ref, w1e_ref, w1d_ref, b1_ref,
              w2_ref, b2_ref, wout_ref, o_ref):
    hi = jax.lax.Precision.HIGHEST
    xT = xT_ref[...]
    embT = embT_ref[...]
    xdT = xT[N_SPARSE:, :]

    wide = jnp.maximum(
        jnp.dot(wlin_ref[...], xT, precision=hi,
                preferred_element_type=jnp.float32) + blin_ref[...], 0.0)

    h1 = jnp.dot(w1e_ref[...], embT, precision=hi,
                 preferred_element_type=jnp.float32)
    h1 = h1 + jnp.dot(w1d_ref[...], xdT, precision=hi,
                      preferred_element_type=jnp.float32)
    h1 = jnp.maximum(h1 + b1_ref[...], 0.0)

    h2 = jnp.maximum(
        jnp.dot(w2_ref[...], h1, precision=hi,
                preferred_element_type=jnp.float32) + b2_ref[...], 0.0)

    z = wide + jnp.dot(wout_ref[...], h2, precision=hi,
                       preferred_element_type=jnp.float32)
    o_ref[...] = 1.0 / (1.0 + jnp.exp(-z))


def _mlp(embT, XT, wlin, blin, w1e, w1d, b1, w2, b2, wout):
    full = lambda shape: pl.BlockSpec(shape, lambda i: (0, 0))
    return pl.pallas_call(
        _mlp_body,
        grid=(B // TB,),
        in_specs=[
            pl.BlockSpec((SPARSE_DIM, TB), lambda i: (0, i)),
            pl.BlockSpec((N_FEAT, TB), lambda i: (0, i)),
            full((1, N_FEAT)),
            full((1, 1)),
            full((HID1, SPARSE_DIM)),
            full((HID1, N_DENSE)),
            full((HID1, 1)),
            full((HID2, HID1)),
            full((HID2, 1)),
            full((1, HID2)),
        ],
        out_specs=pl.BlockSpec((1, TB), lambda i: (0, i)),
        out_shape=jax.ShapeDtypeStruct((1, B), jnp.float32),
    )(embT, XT, wlin, blin, w1e, w1d, b1, w2, b2, wout)


def kernel(X, tables, W_lin, b_lin, W1, b1, W2, b2, W_out):
    idx_t = X[:, :N_SPARSE].astype(jnp.int32).T
    tab2 = jnp.swapaxes(tables, 1, 2).reshape(SPARSE_DIM, VOCAB)

    embT = _sc_gather(idx_t, tab2)

    XT = X.T
    y = _mlp(embT, XT,
             W_lin, b_lin.reshape(1, 1),
             W1[:, :SPARSE_DIM], W1[:, SPARSE_DIM:], b1.reshape(HID1, 1),
             W2, b2.reshape(HID2, 1),
             W_out)
    return y.reshape(B, 1)


# speedup vs baseline: 6.0107x; 1.1460x over previous
import jax
import jax.numpy as jnp
from jax import lax
from jax.experimental import pallas as pl
from jax.experimental.pallas import tpu as pltpu
from jax.experimental.pallas import tpu_sc as plsc

B = 16384
N_SPARSE = 26
N_DENSE = 13
VOCAB = 100000
EDIM = 16
HID1 = 256
HID2 = 128
N_FEAT = N_SPARSE + N_DENSE
SPARSE_DIM = N_SPARSE * EDIM

NC = 2
NSUB = 16
NW = NC * NSUB
PPW = SPARSE_DIM // NW
ICH = 2048
NICH = B // ICH

TB = 1024


def _sc_gather_body(idx_hbm, tab_hbm, out_hbm, vec_v, idx_v, row_v,
                    ssem, osem0, osem1):
    wid = lax.axis_index("s") * NC + lax.axis_index("c")
    osems = (osem0, osem1)

    def per_pair(k, carry):
        p = wid * PPW + k
        f = p // EDIM
        pltpu.async_copy(tab_hbm.at[p], vec_v, ssem).wait()

        def per_block_pair(jj, carry2):
            for s in range(2):
                j = jj * 2 + s
                b0 = j * ICH
                pltpu.sync_copy(idx_hbm.at[f, pl.ds(b0, ICH)], idx_v.at[s])

                @plsc.parallel_loop(0, ICH, step=16, unroll=8)
                def _gather(i, s=s):
                    sl = pl.ds(i, 16)
                    iv = idx_v[s, sl]
                    row_v[s, sl] = plsc.load_gather(vec_v, [iv])

                @pl.when(j >= 2)
                def _(s=s, b0=b0):
                    pltpu.make_async_copy(
                        row_v.at[s], out_hbm.at[p, pl.ds(b0 - 2 * ICH, ICH)],
                        osems[s]).wait()
                pltpu.async_copy(row_v.at[s], out_hbm.at[p, pl.ds(b0, ICH)],
                                 osems[s])
            return carry2
        lax.fori_loop(0, NICH // 2, per_block_pair, 0)

        for jb in range(2):
            b_last = (NICH - 2 + jb) * ICH
            pltpu.make_async_copy(
                row_v.at[jb], out_hbm.at[p, pl.ds(b_last, ICH)],
                osems[jb]).wait()
        return carry
    lax.fori_loop(0, PPW, per_pair, 0)


def _sc_gather(idx_t, tab2):
    mesh = plsc.VectorSubcoreMesh(core_axis_name="c", subcore_axis_name="s",
                                  num_cores=NC, num_subcores=NSUB)
    f = pl.kernel(
        _sc_gather_body,
        out_type=jax.ShapeDtypeStruct((SPARSE_DIM, B), jnp.float32),
        mesh=mesh,
        compiler_params=pltpu.CompilerParams(use_tc_tiling_on_sc=True,
                                             needs_layout_passes=False),
        scratch_types=[
            pltpu.VMEM((VOCAB,), jnp.float32),
            pltpu.VMEM((2, ICH), jnp.int32),
            pltpu.VMEM((2, ICH), jnp.float32),
            pltpu.SemaphoreType.DMA,
            pltpu.SemaphoreType.DMA,
            pltpu.SemaphoreType.DMA,
        ],
    )
    return f(idx_t, tab2)


def _mlp_body(embT_ref, xT_ref, wlin_ref, blin_ref, w1e_ref, w1d_ref, b1_ref,
              w2_ref, b2_ref, wout_ref, o_ref):
    hi = jax.lax.Precision.HIGHEST
    xT = xT_ref[...]
    embT = embT_ref[...]
    xdT = xT[N_SPARSE:, :]

    wide = jnp.maximum(
        jnp.dot(wlin_ref[...], xT, precision=hi,
                preferred_element_type=jnp.float32) + blin_ref[...], 0.0)

    h1 = jnp.dot(w1e_ref[...], embT,
                 preferred_element_type=jnp.float32)
    h1 = h1 + jnp.dot(w1d_ref[...], xdT, precision=hi,
                      preferred_element_type=jnp.float32)
    h1 = jnp.maximum(h1 + b1_ref[...], 0.0)

    h2 = jnp.maximum(
        jnp.dot(w2_ref[...], h1,
                preferred_element_type=jnp.float32) + b2_ref[...], 0.0)

    z = wide + jnp.dot(wout_ref[...], h2,
                       preferred_element_type=jnp.float32)
    o_ref[...] = 1.0 / (1.0 + jnp.exp(-z))


def _mlp(embT, XT, wlin, blin, w1e, w1d, b1, w2, b2, wout):
    full = lambda shape: pl.BlockSpec(shape, lambda i: (0, 0))
    return pl.pallas_call(
        _mlp_body,
        grid=(B // TB,),
        in_specs=[
            pl.BlockSpec((SPARSE_DIM, TB), lambda i: (0, i)),
            pl.BlockSpec((N_FEAT, TB), lambda i: (0, i)),
            full((1, N_FEAT)),
            full((1, 1)),
            full((HID1, SPARSE_DIM)),
            full((HID1, N_DENSE)),
            full((HID1, 1)),
            full((HID2, HID1)),
            full((HID2, 1)),
            full((1, HID2)),
        ],
        out_specs=pl.BlockSpec((1, TB), lambda i: (0, i)),
        out_shape=jax.ShapeDtypeStruct((1, B), jnp.float32),
    )(embT, XT, wlin, blin, w1e, w1d, b1, w2, b2, wout)


def kernel(X, tables, W_lin, b_lin, W1, b1, W2, b2, W_out):
    idx_t = X[:, :N_SPARSE].astype(jnp.int32).T
    tab2 = jnp.swapaxes(tables, 1, 2).reshape(SPARSE_DIM, VOCAB)

    embT = _sc_gather(idx_t, tab2)

    XT = X.T
    y = _mlp(embT, XT,
             W_lin, b_lin.reshape(1, 1),
             W1[:, :SPARSE_DIM], W1[:, SPARSE_DIM:], b1.reshape(HID1, 1),
             W2, b2.reshape(HID2, 1),
             W_out)
    return y.reshape(B, 1)


# speedup vs baseline: 6.0391x; 1.0047x over previous
import jax
import jax.numpy as jnp
from jax import lax
from jax.experimental import pallas as pl
from jax.experimental.pallas import tpu as pltpu
from jax.experimental.pallas import tpu_sc as plsc

B = 16384
N_SPARSE = 26
N_DENSE = 13
VOCAB = 100000
EDIM = 16
HID1 = 256
HID2 = 128
N_FEAT = N_SPARSE + N_DENSE
SPARSE_DIM = N_SPARSE * EDIM

NC = 2
NSUB = 16
NW = NC * NSUB
PPW = SPARSE_DIM // NW
ICH = 2048
NICH = B // ICH

TB = 1024


def _sc_gather_body(idx_hbm, tab_hbm, out_hbm, vec_v, idx_v, row_v,
                    ssem, osem0, osem1):
    wid = lax.axis_index("s") * NC + lax.axis_index("c")
    osems = (osem0, osem1)

    def per_pair(k, carry):
        p = wid * PPW + k
        f = p // EDIM
        pltpu.async_copy(tab_hbm.at[p], vec_v, ssem).wait()

        def per_block_pair(jj, carry2):
            for s in range(2):
                j = jj * 2 + s
                b0 = j * ICH
                pltpu.sync_copy(idx_hbm.at[f, pl.ds(b0, ICH)], idx_v.at[s])

                @plsc.parallel_loop(0, ICH, step=16, unroll=16)
                def _gather(i, s=s):
                    sl = pl.ds(i, 16)
                    iv = idx_v[s, sl]
                    row_v[s, sl] = plsc.load_gather(vec_v, [iv])

                @pl.when(j >= 2)
                def _(s=s, b0=b0):
                    pltpu.make_async_copy(
                        row_v.at[s], out_hbm.at[p, pl.ds(b0 - 2 * ICH, ICH)],
                        osems[s]).wait()
                pltpu.async_copy(row_v.at[s], out_hbm.at[p, pl.ds(b0, ICH)],
                                 osems[s])
            return carry2
        lax.fori_loop(0, NICH // 2, per_block_pair, 0)

        for jb in range(2):
            b_last = (NICH - 2 + jb) * ICH
            pltpu.make_async_copy(
                row_v.at[jb], out_hbm.at[p, pl.ds(b_last, ICH)],
                osems[jb]).wait()
        return carry
    lax.fori_loop(0, PPW, per_pair, 0)


def _sc_gather(idx_t, tab2):
    mesh = plsc.VectorSubcoreMesh(core_axis_name="c", subcore_axis_name="s",
                                  num_cores=NC, num_subcores=NSUB)
    f = pl.kernel(
        _sc_gather_body,
        out_type=jax.ShapeDtypeStruct((SPARSE_DIM, B), jnp.float32),
        mesh=mesh,
        compiler_params=pltpu.CompilerParams(use_tc_tiling_on_sc=True,
                                             needs_layout_passes=False),
        scratch_types=[
            pltpu.VMEM((VOCAB,), jnp.float32),
            pltpu.VMEM((2, ICH), jnp.int32),
            pltpu.VMEM((2, ICH), jnp.float32),
            pltpu.SemaphoreType.DMA,
            pltpu.SemaphoreType.DMA,
            pltpu.SemaphoreType.DMA,
        ],
    )
    return f(idx_t, tab2)


def _mlp_body(embT_ref, xT_ref, wlin_ref, blin_ref, w1e_ref, w1d_ref, b1_ref,
              w2_ref, b2_ref, wout_ref, o_ref):
    hi = jax.lax.Precision.HIGHEST
    xT = xT_ref[...]
    embT = embT_ref[...]
    xdT = xT[N_SPARSE:, :]

    wide = jnp.maximum(
        jnp.dot(wlin_ref[...], xT, precision=hi,
                preferred_element_type=jnp.float32) + blin_ref[...], 0.0)

    h1 = jnp.dot(w1e_ref[...], embT,
                 preferred_element_type=jnp.float32)
    h1 = h1 + jnp.dot(w1d_ref[...], xdT, precision=hi,
                      preferred_element_type=jnp.float32)
    h1 = jnp.maximum(h1 + b1_ref[...], 0.0)

    h2 = jnp.maximum(
        jnp.dot(w2_ref[...], h1,
                preferred_element_type=jnp.float32) + b2_ref[...], 0.0)

    z = wide + jnp.dot(wout_ref[...], h2,
                       preferred_element_type=jnp.float32)
    o_ref[...] = 1.0 / (1.0 + jnp.exp(-z))


def _mlp(embT, XT, wlin, blin, w1e, w1d, b1, w2, b2, wout):
    full = lambda shape: pl.BlockSpec(shape, lambda i: (0, 0))
    return pl.pallas_call(
        _mlp_body,
        grid=(B // TB,),
        in_specs=[
            pl.BlockSpec((SPARSE_DIM, TB), lambda i: (0, i)),
            pl.BlockSpec((N_FEAT, TB), lambda i: (0, i)),
            full((1, N_FEAT)),
            full((1, 1)),
            full((HID1, SPARSE_DIM)),
            full((HID1, N_DENSE)),
            full((HID1, 1)),
            full((HID2, HID1)),
            full((HID2, 1)),
            full((1, HID2)),
        ],
        out_specs=pl.BlockSpec((1, TB), lambda i: (0, i)),
        out_shape=jax.ShapeDtypeStruct((1, B), jnp.float32),
    )(embT, XT, wlin, blin, w1e, w1d, b1, w2, b2, wout)


def kernel(X, tables, W_lin, b_lin, W1, b1, W2, b2, W_out):
    idx_t = X[:, :N_SPARSE].astype(jnp.int32).T
    tab2 = jnp.swapaxes(tables, 1, 2).reshape(SPARSE_DIM, VOCAB)

    embT = _sc_gather(idx_t, tab2)

    XT = X.T
    y = _mlp(embT, XT,
             W_lin, b_lin.reshape(1, 1),
             W1[:, :SPARSE_DIM], W1[:, SPARSE_DIM:], b1.reshape(HID1, 1),
             W2, b2.reshape(HID2, 1),
             W_out)
    return y.reshape(B, 1)
